# Initial kernel scaffold; baseline (speedup 1.0000x reference)
import jax
import jax.numpy as jnp
from jax.experimental import pallas as pl


def kernel(z, codebooks):
    raise NotImplementedError("write your pallas kernel here")


# speedup vs baseline: 3.8756x; 3.8756x over previous
import functools

import jax
import jax.numpy as jnp
from jax import lax
from jax.experimental import pallas as pl
from jax.experimental.pallas import tpu as pltpu
from jax.experimental.pallas import tpu_sc as plsc

B, H, K, D = 8192, 4, 1024, 256
BM = 512
GRID = B // BM

ROWS = B * H
NW = 32
RPW = ROWS // NW
CH = 128
NCH = RPW // CH


def _tc_body(z_ref, cb_ref, idx_ref, flat_ref, loss_ref):
    i = pl.program_id(0)

    @pl.when(i == 0)
    def _init():
        loss_ref[...] = jnp.zeros_like(loss_ref)

    total = jnp.float32(0.0)
    for h in range(H):
        zh = z_ref[:, h * D:(h + 1) * D]
        ch = cb_ref[h]
        cross = lax.dot_general(
            zh, ch, (((1,), (1,)), ((), ())),
            preferred_element_type=jnp.float32)
        z2 = jnp.sum(zh * zh, axis=1, keepdims=True)
        c2 = jnp.sum(ch * ch, axis=1)[None, :]
        dist = (z2 + c2) - 2.0 * cross
        m = jnp.min(dist, axis=1, keepdims=True)
        iota = lax.broadcasted_iota(jnp.int32, (BM, K), 1)
        idxh = jnp.min(jnp.where(dist == m, iota, K), axis=1, keepdims=True)
        idx_ref[:, h:h + 1] = idxh
        flat_ref[:, h:h + 1] = idxh + h * K
        total = total + jnp.sum(m)

    loss_ref[...] += jnp.broadcast_to(total * (1.0 / 1024.0), (8, 128))


@functools.cache
def _make_sc_gather():
    mesh = plsc.VectorSubcoreMesh(core_axis_name="c", subcore_axis_name="s")

    @functools.partial(
        pl.kernel,
        mesh=mesh,
        out_type=jax.ShapeDtypeStruct((ROWS, D), jnp.float32),
        scratch_types=[
            pltpu.VMEM((CH,), jnp.int32),
            pltpu.VMEM((CH, D), jnp.float32),
            pltpu.SemaphoreType.DMA,
        ],
    )
    def _sc_gather(table_hbm, idx_hbm, out_hbm, idx_v, rows_v, sem):
        wid = lax.axis_index("s") * 2 + lax.axis_index("c")
        base = wid * RPW
        for c in range(NCH):
            off = base + c * CH
            pltpu.sync_copy(idx_hbm.at[pl.ds(off, CH)], idx_v)
            pltpu.async_copy(table_hbm.at[idx_v], rows_v, sem).wait()
            pltpu.sync_copy(rows_v, out_hbm.at[pl.ds(off, CH)])

    return _sc_gather


def kernel(z, codebooks):
    idx, flat, loss_part = pl.pallas_call(
        _tc_body,
        grid=(GRID,),
        in_specs=[
            pl.BlockSpec((BM, H * D), lambda i: (i, 0)),
            pl.BlockSpec((H, K, D), lambda i: (0, 0, 0)),
        ],
        out_specs=[
            pl.BlockSpec((BM, H), lambda i: (i, 0)),
            pl.BlockSpec((BM, H), lambda i: (i, 0)),
            pl.BlockSpec((8, 128), lambda i: (0, 0)),
        ],
        out_shape=[
            jax.ShapeDtypeStruct((B, H), jnp.int32),
            jax.ShapeDtypeStruct((B, H), jnp.int32),
            jax.ShapeDtypeStruct((8, 128), jnp.float32),
        ],
    )(z, codebooks)

    table = codebooks.reshape(H * K, D)
    zq_flat = _make_sc_gather()(table, flat.reshape(ROWS))
    z_q = zq_flat.reshape(B, H * D)
    commit_loss = jnp.sum(loss_part) * (1.0 / (B * H * D))
    return z_q, idx, commit_loss


# speedup vs baseline: 5.1723x; 1.3346x over previous
import functools

import jax
import jax.numpy as jnp
from jax import lax
from jax.experimental import pallas as pl
from jax.experimental.pallas import tpu as pltpu
from jax.experimental.pallas import tpu_sc as plsc

B, H, K, D = 8192, 4, 1024, 256
BM = 512
GRID = B // BM

ROWS = B * H
NW = 32
RPW = ROWS // NW
CH = 128
NCH = RPW // CH


def _tc_body(z_ref, cb_ref, idx_ref, flat_ref, loss_ref):
    i = pl.program_id(0)

    @pl.when(i == 0)
    def _init():
        loss_ref[...] = jnp.zeros_like(loss_ref)

    total = jnp.float32(0.0)
    for h in range(H):
        zh = z_ref[:, h * D:(h + 1) * D]
        ch = cb_ref[h]
        cross = lax.dot_general(
            zh, ch, (((1,), (1,)), ((), ())),
            preferred_element_type=jnp.float32)
        z2 = jnp.sum(zh * zh, axis=1, keepdims=True)
        c2 = jnp.sum(ch * ch, axis=1)[None, :]
        dist = (z2 + c2) - 2.0 * cross
        m = jnp.min(dist, axis=1, keepdims=True)
        iota = lax.broadcasted_iota(jnp.int32, (BM, K), 1)
        idxh = jnp.min(jnp.where(dist == m, iota, K), axis=1, keepdims=True)
        idx_ref[:, h:h + 1] = idxh
        flat_ref[:, h:h + 1] = idxh + h * K
        total = total + jnp.sum(m)

    loss_ref[...] += jnp.broadcast_to(total * (1.0 / 1024.0), (8, 128))


@functools.cache
def _make_sc_gather():
    mesh = plsc.VectorSubcoreMesh(core_axis_name="c", subcore_axis_name="s")

    @functools.partial(
        pl.kernel,
        mesh=mesh,
        out_type=jax.ShapeDtypeStruct((B, H * D), jnp.float32),
        scratch_types=[
            pltpu.VMEM((CH,), jnp.int32),
            pltpu.VMEM((CH,), jnp.int32),
            pltpu.VMEM((CH, D), jnp.float32),
            pltpu.VMEM((CH, D), jnp.float32),
            pltpu.SemaphoreType.DMA,
            pltpu.SemaphoreType.DMA,
            pltpu.SemaphoreType.DMA,
            pltpu.SemaphoreType.DMA,
        ],
    )
    def _sc_gather(table_hbm, idxt_hbm, out_hbm,
                   idx_a, idx_b, rows_a, rows_b, gs_a, gs_b, os_a, os_b):
        wid = lax.axis_index("s") * 2 + lax.axis_index("c")
        h = wid // 8
        tok0 = (wid % 8) * RPW
        ibase = h * B + tok0
        col = h * D
        idx_v = (idx_a, idx_b)
        rows_v = (rows_a, rows_b)
        gsem = (gs_a, gs_b)
        osem = (os_a, os_b)
        gathers = [None, None]
        writes = [None, None]
        pltpu.sync_copy(idxt_hbm.at[pl.ds(ibase, CH)], idx_a)
        gathers[0] = pltpu.async_copy(table_hbm.at[idx_a], rows_a, gs_a)
        for c in range(NCH):
            p = c % 2
            q = (c + 1) % 2
            gathers[p].wait()
            writes[p] = pltpu.async_copy(
                rows_v[p],
                out_hbm.at[pl.ds(tok0 + c * CH, CH), pl.ds(col, D)],
                osem[p])
            if c + 1 < NCH:
                pltpu.sync_copy(
                    idxt_hbm.at[pl.ds(ibase + (c + 1) * CH, CH)], idx_v[q])
                if writes[q] is not None:
                    writes[q].wait()
                gathers[q] = pltpu.async_copy(
                    table_hbm.at[idx_v[q]], rows_v[q], gsem[q])
        writes[0].wait()
        writes[1].wait()

    return _sc_gather


def kernel(z, codebooks):
    idx, flat, loss_part = pl.pallas_call(
        _tc_body,
        grid=(GRID,),
        in_specs=[
            pl.BlockSpec((BM, H * D), lambda i: (i, 0)),
            pl.BlockSpec((H, K, D), lambda i: (0, 0, 0)),
        ],
        out_specs=[
            pl.BlockSpec((BM, H), lambda i: (i, 0)),
            pl.BlockSpec((BM, H), lambda i: (i, 0)),
            pl.BlockSpec((8, 128), lambda i: (0, 0)),
        ],
        out_shape=[
            jax.ShapeDtypeStruct((B, H), jnp.int32),
            jax.ShapeDtypeStruct((B, H), jnp.int32),
            jax.ShapeDtypeStruct((8, 128), jnp.float32),
        ],
    )(z, codebooks)

    table = codebooks.reshape(H * K, D)
    flat_t = flat.T.reshape(ROWS)
    z_q = _make_sc_gather()(table, flat_t)
    commit_loss = jnp.sum(loss_part) * (1.0 / (B * H * D))
    return z_q, idx, commit_loss
